# baseline (speedup 1.0000x reference)
import functools

import jax
import jax.numpy as jnp
from jax.experimental import pallas as pl
from jax.experimental.pallas import tpu as pltpu

_BN = 256


def _dist_kernel(x_ref, e_ref, o_ref, xa_ref, ea_ref):
    i = pl.program_id(0)
    bn = o_ref.shape[0]

    @pl.when(i == 0)
    def _build_aug():
        e = e_ref[...]
        ee = jnp.sum(e * e, axis=1, keepdims=True)
        ea_ref[...] = jnp.concatenate(
            [e, jnp.ones((e.shape[0], 1), jnp.float32), ee], axis=1)
        x = x_ref[...]
        xx = jnp.sum(x * x, axis=1, keepdims=True)
        xa_ref[...] = jnp.concatenate(
            [-2.0 * x, xx, jnp.ones((x.shape[0], 1), jnp.float32)], axis=1)

    xa = xa_ref[pl.ds(i * bn, bn), :]
    d2 = jax.lax.dot_general(
        xa, ea_ref[...], (((1,), (1,)), ((), ())),
        preferred_element_type=jnp.float32)
    d2 = jnp.maximum(d2, 1e-36)
    o_ref[...] = d2 * jax.lax.rsqrt(d2)


@functools.partial(jax.jit, static_argnames=())
def kernel(inputs, embeddings):
    n, d = inputs.shape
    k, _ = embeddings.shape
    return pl.pallas_call(
        _dist_kernel,
        grid=(n // _BN,),
        in_specs=[
            pl.BlockSpec((n, d), lambda i: (0, 0)),
            pl.BlockSpec((k, d), lambda i: (0, 0)),
        ],
        out_specs=pl.BlockSpec((_BN, k), lambda i: (i, 0)),
        out_shape=jax.ShapeDtypeStruct((n, k), jnp.float32),
        scratch_shapes=[
            pltpu.VMEM((n, d + 2), jnp.float32),
            pltpu.VMEM((k, d + 2), jnp.float32),
        ],
        compiler_params=pltpu.CompilerParams(
            dimension_semantics=("arbitrary",),
        ),
    )(inputs, embeddings)


# speedup vs baseline: 1.0008x; 1.0008x over previous
import functools

import jax
import jax.numpy as jnp
from jax.experimental import pallas as pl
from jax.experimental.pallas import tpu as pltpu

_BN = 512


def _dist_kernel(x_ref, e_ref, o_ref, ea_ref):
    bk = e_ref.shape[0]

    @pl.when(pl.program_id(0) == 0)
    def _build_e_aug():
        e = e_ref[...]
        ee = jnp.sum(e * e, axis=1, keepdims=True)
        ea_ref[...] = jnp.concatenate(
            [e, jnp.ones((bk, 1), jnp.float32), ee], axis=1)

    x = x_ref[...]
    xx = jnp.sum(x * x, axis=1, keepdims=True)
    xa = jnp.concatenate(
        [-2.0 * x, xx, jnp.ones((x.shape[0], 1), jnp.float32)], axis=1)
    d2 = jax.lax.dot_general(
        xa, ea_ref[...], (((1,), (1,)), ((), ())),
        preferred_element_type=jnp.float32)
    d2 = jnp.maximum(d2, 1e-36)
    o_ref[...] = d2 * jax.lax.rsqrt(d2)


@functools.partial(jax.jit, static_argnames=())
def kernel(inputs, embeddings):
    n, d = inputs.shape
    k, _ = embeddings.shape
    return pl.pallas_call(
        _dist_kernel,
        grid=(n // _BN,),
        in_specs=[
            pl.BlockSpec((_BN, d), lambda i: (i, 0)),
            pl.BlockSpec((k, d), lambda i: (0, 0)),
        ],
        out_specs=pl.BlockSpec((_BN, k), lambda i: (i, 0)),
        out_shape=jax.ShapeDtypeStruct((n, k), jnp.float32),
        scratch_shapes=[pltpu.VMEM((k, d + 2), jnp.float32)],
        compiler_params=pltpu.CompilerParams(
            dimension_semantics=("arbitrary",),
        ),
    )(inputs, embeddings)
